# Initial kernel scaffold; baseline (speedup 1.0000x reference)
import jax
import jax.numpy as jnp
from jax.experimental import pallas as pl


def kernel(x, routing, W, b, gamma, beta, use_boundary):
    raise NotImplementedError("write your pallas kernel here")


# speedup vs baseline: 2.0840x; 2.0840x over previous
import jax
import jax.numpy as jnp
from jax.experimental import pallas as pl


_LANES = 128
_EPS = 1e-5


def _tiny_block_kernel(x_ref, s_ref, g_ref, c_ref, o_ref):
    x = x_ref[...]
    s = s_ref[...]
    dims = (((1,), (0,)), ((), ()))
    mu = jax.lax.dot_general(x, s, dims, preferred_element_type=jnp.float32)
    d = x - mu
    var = jax.lax.dot_general(d * d, s, dims, preferred_element_type=jnp.float32)
    hn = d * jax.lax.rsqrt(var + _EPS)
    y = jax.lax.dot_general(hn, g_ref[...], dims,
                            preferred_element_type=jnp.float32)
    o_ref[...] = x + y + c_ref[...]


def kernel(x, routing, W, b, gamma, beta, use_boundary):
    n, h = x.shape
    grp = _LANES // h
    rows = n // grp

    x2 = x.reshape(rows, _LANES)

    eye = jnp.eye(grp, dtype=jnp.float32)
    s = jnp.kron(eye, jnp.full((h, h), 1.0 / h, dtype=jnp.float32))
    g = jnp.kron(eye, gamma[:, None] * W.T)
    c = jnp.tile(beta @ W.T + b, (grp,)).reshape(1, _LANES)

    out2 = pl.pallas_call(
        _tiny_block_kernel,
        out_shape=jax.ShapeDtypeStruct((rows, _LANES), jnp.float32),
    )(x2, s, g, c)

    return out2.reshape(n, h)


# speedup vs baseline: 3.3118x; 1.5892x over previous
import jax
import jax.numpy as jnp
from jax.experimental import pallas as pl


_EPS = 1e-5
_NUM_BLOCKS = 8


def _tiny_block_kernel(x_ref, a_ref, g_ref, c_ref, o_ref):
    x = x_ref[...]
    a = a_ref[...]
    dims = (((1,), (0,)), ((), ()))
    mu = jax.lax.dot_general(x, a, dims, preferred_element_type=jnp.float32)
    d = x - mu
    var = jax.lax.dot_general(d * d, a, dims,
                              preferred_element_type=jnp.float32)
    hn = d * jax.lax.rsqrt(var + _EPS)
    y = jax.lax.dot_general(hn, g_ref[...], dims,
                            preferred_element_type=jnp.float32)
    o_ref[...] = x + y + c_ref[...]


def kernel(x, routing, W, b, gamma, beta, use_boundary):
    n, h = x.shape
    bm = n // _NUM_BLOCKS

    a = jnp.full((h, h), 1.0 / h, dtype=jnp.float32)
    g = gamma[:, None] * W.T
    c = (beta @ W.T + b).reshape(1, h)

    return pl.pallas_call(
        _tiny_block_kernel,
        grid=(_NUM_BLOCKS,),
        in_specs=[
            pl.BlockSpec((bm, h), lambda i: (i, 0)),
            pl.BlockSpec((h, h), lambda i: (0, 0)),
            pl.BlockSpec((h, h), lambda i: (0, 0)),
            pl.BlockSpec((1, h), lambda i: (0, 0)),
        ],
        out_specs=pl.BlockSpec((bm, h), lambda i: (i, 0)),
        out_shape=jax.ShapeDtypeStruct((n, h), jnp.float32),
    )(x, a, g, c)
